# baseline (speedup 1.0000x reference)
import jax
import jax.numpy as jnp
from jax import lax
from jax.experimental import pallas as pl
from jax.experimental.pallas import tpu as pltpu

N = 320000
D = 128
NUM_SEG = 10000

R = 512
NBLK = N // R
S = 128
NW = (NUM_SEG + S - 1) // S
SEGP = NW * S
CHUNKS = NBLK + 2 * NW


def _mlp_seg_kernel(rb_ref, w_ref, fl_ref,
                    x_ref, idx_ref, W1_ref, b1_ref, W2_ref, b2_ref,
                    out_ref):
    g = pl.program_id(0)
    flag = fl_ref[g]
    w = w_ref[g]

    xb = x_ref[...].astype(jnp.bfloat16)
    h = jnp.dot(xb, W1_ref[...], preferred_element_type=jnp.float32)
    h = jnp.maximum(h + b1_ref[...], 0.0).astype(jnp.bfloat16)
    t = jnp.dot(h, W2_ref[...], preferred_element_type=jnp.float32)
    t = jnp.maximum(t + b2_ref[...], 0.0).astype(jnp.bfloat16)

    local = idx_ref[0, 0, :] - w * S
    local = jnp.where(flag >= 0, local, -1)
    iota = lax.broadcasted_iota(jnp.int32, (S, R), 0)
    oh = (iota == local[None, :]).astype(jnp.bfloat16)
    part = jnp.dot(oh, t, preferred_element_type=jnp.float32)

    @pl.when(flag == 1)
    def _():
        out_ref[...] = part

    @pl.when(flag != 1)
    def _():
        out_ref[...] += part


def _head_kernel(seg_ref, W3_ref, b3_ref, W4_ref, b4_ref, out_ref):
    u = jnp.dot(seg_ref[...], W3_ref[...], preferred_element_type=jnp.float32)
    u = jnp.maximum(u + b3_ref[...], 0.0)
    v = jnp.dot(u, W4_ref[...], preferred_element_type=jnp.float32)
    out_ref[...] = v + b4_ref[...]


def kernel(x, idx, W1, b1, W2, b2, W3, b3, W4, b4):
    idx32 = idx.astype(jnp.int32)

    bounds = jnp.arange(NW + 1, dtype=jnp.int32) * S
    starts = jnp.searchsorted(idx32, bounds, side="left").astype(jnp.int32)
    b0 = starts[:-1] // R
    e = (starts[1:] + R - 1) // R
    nch = jnp.maximum(e - b0, 1)
    csum = jnp.cumsum(nch)
    cid = jnp.arange(CHUNKS, dtype=jnp.int32)
    wofc = jnp.searchsorted(csum, cid, side="right").astype(jnp.int32)
    valid = wofc < NW
    wc = jnp.minimum(wofc, NW - 1)
    offs = csum - nch
    local = cid - offs[wc]
    rb = jnp.clip(b0[wc] + local, 0, NBLK - 1).astype(jnp.int32)
    flag = jnp.where(valid, jnp.where(local == 0, 1, 0), -1).astype(jnp.int32)

    idx3 = idx32.reshape(NBLK, 1, R)
    b1r = b1.reshape(1, D)
    b2r = b2.reshape(1, D)

    seg = pl.pallas_call(
        _mlp_seg_kernel,
        grid_spec=pltpu.PrefetchScalarGridSpec(
            num_scalar_prefetch=3,
            grid=(CHUNKS,),
            in_specs=[
                pl.BlockSpec((R, D), lambda g, rb, w, fl: (rb[g], 0)),
                pl.BlockSpec((1, 1, R), lambda g, rb, w, fl: (rb[g], 0, 0)),
                pl.BlockSpec((D, D), lambda g, rb, w, fl: (0, 0)),
                pl.BlockSpec((1, D), lambda g, rb, w, fl: (0, 0)),
                pl.BlockSpec((D, D), lambda g, rb, w, fl: (0, 0)),
                pl.BlockSpec((1, D), lambda g, rb, w, fl: (0, 0)),
            ],
            out_specs=pl.BlockSpec((S, D), lambda g, rb, w, fl: (w[g], 0)),
        ),
        out_shape=jax.ShapeDtypeStruct((SEGP, D), jnp.float32),
    )(rb, wc, flag, x, idx3, W1.astype(jnp.bfloat16), b1r,
      W2.astype(jnp.bfloat16), b2r)

    out = pl.pallas_call(
        _head_kernel,
        in_specs=[
            pl.BlockSpec((SEGP, D), lambda: (0, 0)),
            pl.BlockSpec((D, 20), lambda: (0, 0)),
            pl.BlockSpec((1, 20), lambda: (0, 0)),
            pl.BlockSpec((20, 1), lambda: (0, 0)),
            pl.BlockSpec((1, 1), lambda: (0, 0)),
        ],
        out_specs=pl.BlockSpec((SEGP, 1), lambda: (0, 0)),
        out_shape=jax.ShapeDtypeStruct((SEGP, 1), jnp.float32),
    )(seg, W3, b3.reshape(1, 20), W4, b4.reshape(1, 1))

    return out[:NUM_SEG]


# speedup vs baseline: 1.0116x; 1.0116x over previous
import jax
import jax.numpy as jnp
from jax import lax
from jax.experimental import pallas as pl
from jax.experimental.pallas import tpu as pltpu

N = 320000
D = 128
NUM_SEG = 10000

R = 512
NBLK = N // R
S = 128
NW = (NUM_SEG + S - 1) // S
SEGP = NW * S
CHUNKS = NBLK + 2 * NW


def _mlp_seg_kernel(rb_ref, w_ref, fl_ref,
                    x_ref, idx_ref, W1_ref, b1_ref, W2_ref, b2_ref,
                    out_ref):
    g = pl.program_id(0)
    flag = fl_ref[g]
    w = w_ref[g]

    h = jnp.dot(x_ref[...], W1_ref[...], preferred_element_type=jnp.float32)
    h = jnp.maximum(h + b1_ref[...], 0.0)
    t = jnp.dot(h, W2_ref[...], preferred_element_type=jnp.float32)
    t = jnp.maximum(t + b2_ref[...], 0.0)

    local = idx_ref[0, 0, :] - w * S
    local = jnp.where(flag >= 0, local, -1)
    iota = lax.broadcasted_iota(jnp.int32, (S, R), 0)
    oh = (iota == local[None, :]).astype(jnp.float32)
    part = jnp.dot(oh, t, preferred_element_type=jnp.float32)

    @pl.when(flag == 1)
    def _():
        out_ref[...] = part

    @pl.when(flag != 1)
    def _():
        out_ref[...] += part


def _head_kernel(seg_ref, W3_ref, b3_ref, W4_ref, b4_ref, out_ref):
    u = jnp.dot(seg_ref[...], W3_ref[...], preferred_element_type=jnp.float32)
    u = jnp.maximum(u + b3_ref[...], 0.0)
    v = jnp.dot(u, W4_ref[...], preferred_element_type=jnp.float32)
    out_ref[...] = v + b4_ref[...]


def kernel(x, idx, W1, b1, W2, b2, W3, b3, W4, b4):
    idx32 = idx.astype(jnp.int32)

    bounds = jnp.arange(NW + 1, dtype=jnp.int32) * S
    starts = jnp.searchsorted(idx32, bounds, side="left").astype(jnp.int32)
    b0 = starts[:-1] // R
    e = (starts[1:] + R - 1) // R
    nch = jnp.maximum(e - b0, 1)
    csum = jnp.cumsum(nch)
    cid = jnp.arange(CHUNKS, dtype=jnp.int32)
    wofc = jnp.searchsorted(csum, cid, side="right").astype(jnp.int32)
    valid = wofc < NW
    wc = jnp.minimum(wofc, NW - 1)
    offs = csum - nch
    local = cid - offs[wc]
    rb = jnp.clip(b0[wc] + local, 0, NBLK - 1).astype(jnp.int32)
    flag = jnp.where(valid, jnp.where(local == 0, 1, 0), -1).astype(jnp.int32)

    idx3 = idx32.reshape(NBLK, 1, R)
    b1r = b1.reshape(1, D)
    b2r = b2.reshape(1, D)

    seg = pl.pallas_call(
        _mlp_seg_kernel,
        grid_spec=pltpu.PrefetchScalarGridSpec(
            num_scalar_prefetch=3,
            grid=(CHUNKS,),
            in_specs=[
                pl.BlockSpec((R, D), lambda g, rb, w, fl: (rb[g], 0)),
                pl.BlockSpec((1, 1, R), lambda g, rb, w, fl: (rb[g], 0, 0)),
                pl.BlockSpec((D, D), lambda g, rb, w, fl: (0, 0)),
                pl.BlockSpec((1, D), lambda g, rb, w, fl: (0, 0)),
                pl.BlockSpec((D, D), lambda g, rb, w, fl: (0, 0)),
                pl.BlockSpec((1, D), lambda g, rb, w, fl: (0, 0)),
            ],
            out_specs=pl.BlockSpec((S, D), lambda g, rb, w, fl: (w[g], 0)),
        ),
        out_shape=jax.ShapeDtypeStruct((SEGP, D), jnp.float32),
    )(rb, wc, flag, x, idx3, W1, b1r, W2, b2r)

    out = pl.pallas_call(
        _head_kernel,
        in_specs=[
            pl.BlockSpec((SEGP, D), lambda: (0, 0)),
            pl.BlockSpec((D, 20), lambda: (0, 0)),
            pl.BlockSpec((1, 20), lambda: (0, 0)),
            pl.BlockSpec((20, 1), lambda: (0, 0)),
            pl.BlockSpec((1, 1), lambda: (0, 0)),
        ],
        out_specs=pl.BlockSpec((SEGP, 1), lambda: (0, 0)),
        out_shape=jax.ShapeDtypeStruct((SEGP, 1), jnp.float32),
    )(seg, W3, b3.reshape(1, 20), W4, b4.reshape(1, 1))

    return out[:NUM_SEG]
